# Initial kernel scaffold; baseline (speedup 1.0000x reference)
import jax
import jax.numpy as jnp
from jax.experimental import pallas as pl


def kernel(x, edge_index, values, W, b):
    raise NotImplementedError("write your pallas kernel here")


# speedup vs baseline: 5.8194x; 5.8194x over previous
import functools

import jax
import jax.numpy as jnp
from jax import lax
from jax.experimental import pallas as pl
from jax.experimental.pallas import tpu as pltpu
from jax.experimental.pallas import tpu_sc as plsc

N = 10000
E = 320000
D = 128

NC = 2
NS = 16
NW = NC * NS
EPW = E // NW
K = 80
NCHUNK = EPW // K
ZR = 80
NZCH = N // ZR


def _sc_spmm(x, src2d, dst2d, values):
    mesh = plsc.VectorSubcoreMesh(core_axis_name="c", subcore_axis_name="s")

    @functools.partial(
        pl.kernel,
        mesh=mesh,
        out_type=jax.ShapeDtypeStruct((NC, N, D), jnp.float32),
        scratch_types=[
            pltpu.VMEM((NCHUNK, K), jnp.int32),
            pltpu.VMEM((NCHUNK, K), jnp.int32),
            pltpu.VMEM((K,), jnp.float32),
            pltpu.VMEM((K, D), jnp.float32),
            pltpu.VMEM_SHARED((N, D), jnp.float32),
            pltpu.SemaphoreType.DMA,
        ],
    )
    def spmm(x_hbm, src_hbm, dst_hbm, vals_hbm, out_hbm,
             src_v, dst_v, vals_v, rows_v, acc, sem):
        c = lax.axis_index("c")
        s = lax.axis_index("s")
        wid = c * NS + s

        pltpu.sync_copy(src_hbm.at[wid], src_v)
        pltpu.sync_copy(dst_hbm.at[wid], dst_v)

        def _zb(i, carry):
            rows_v[i // 8, pl.ds((i % 8) * 16, 16)] = jnp.zeros((16,), jnp.float32)
            return carry
        lax.fori_loop(0, ZR * 8, _zb, 0)
        for t in range(-(-NZCH // NS)):
            q = s + NS * t

            @pl.when(q < NZCH)
            def _():
                pltpu.sync_copy(rows_v, acc.at[pl.ds(q * ZR, ZR)])
        plsc.subcore_barrier()

        def _chunk(ci, carry):
            pltpu.sync_copy(vals_hbm.at[pl.ds(wid * EPW + ci * K, K)], vals_v)
            pltpu.async_copy(x_hbm.at[src_v.at[ci]], rows_v, sem).wait()

            for g in range(K // 16):
                vv = vals_v[pl.ds(g * 16, 16)]
                for j in range(16):
                    val = vv[j]
                    r = g * 16 + j
                    for f in range(D // 16):
                        sl = pl.ds(f * 16, 16)
                        rows_v[r, sl] = rows_v[r, sl] * val

            pltpu.sync_copy(rows_v, acc.at[dst_v.at[ci]], add=True)
            return carry
        lax.fori_loop(0, NCHUNK, _chunk, 0)

        plsc.subcore_barrier()

        for t in range(-(-NZCH // NS)):
            q = s + NS * t

            @pl.when(q < NZCH)
            def _():
                pltpu.sync_copy(acc.at[pl.ds(q * ZR, ZR)],
                                out_hbm.at[c, pl.ds(q * ZR, ZR)])

    return spmm(x, src2d, dst2d, values)


BLK = 1000


def _tc_linear_body(acc_ref, w_ref, b_ref, o_ref):
    a = acc_ref[0] + acc_ref[1]
    o_ref[...] = lax.dot_general(
        a, w_ref[...], (((1,), (1,)), ((), ())),
        preferred_element_type=jnp.float32) + b_ref[...]


def _tc_linear(acc2, W, b2):
    return pl.pallas_call(
        _tc_linear_body,
        grid=(N // BLK,),
        in_specs=[
            pl.BlockSpec((NC, BLK, D), lambda i: (0, i, 0)),
            pl.BlockSpec((D, D), lambda i: (0, 0)),
            pl.BlockSpec((1, D), lambda i: (0, 0)),
        ],
        out_specs=pl.BlockSpec((BLK, D), lambda i: (i, 0)),
        out_shape=jax.ShapeDtypeStruct((N, D), jnp.float32),
    )(acc2, W, b2)


def kernel(x, edge_index, values, W, b):
    ei = edge_index.astype(jnp.int32)
    dst2d = ei[0].reshape(NW, NCHUNK, K)
    src2d = ei[1].reshape(NW, NCHUNK, K)
    acc2 = _sc_spmm(x, src2d, dst2d, values)
    return _tc_linear(acc2, W, b.reshape(1, D))


# speedup vs baseline: 11.5603x; 1.9865x over previous
import functools

import jax
import jax.numpy as jnp
from jax import lax
from jax.experimental import pallas as pl
from jax.experimental.pallas import tpu as pltpu
from jax.experimental.pallas import tpu_sc as plsc

N = 10000
E = 320000
D = 128

NC = 2
NS = 16
NW = NC * NS
EPW = E // NW
K = 80
NCHUNK = EPW // K
NR = 4
NI = 8
LA = 4
STEP = 8
ZR = 80
NZCH = N // ZR


def _sc_spmm(x, src1d, dst1d, values):
    mesh = plsc.VectorSubcoreMesh(core_axis_name="c", subcore_axis_name="s")

    @functools.partial(
        pl.kernel,
        mesh=mesh,
        out_type=jax.ShapeDtypeStruct((NC, N, D), jnp.float32),
        scratch_types=[
            pltpu.VMEM((NI, K), jnp.int32),
            pltpu.VMEM((NI, K), jnp.int32),
            pltpu.VMEM((NI, K), jnp.float32),
            pltpu.VMEM((NR, K, D), jnp.float32),
            pltpu.VMEM_SHARED((N, D), jnp.float32),
            pltpu.SemaphoreType.DMA((NI,)),
            pltpu.SemaphoreType.DMA((NR,)),
            pltpu.SemaphoreType.DMA((NR,)),
        ],
    )
    def spmm(x_hbm, src_hbm, dst_hbm, vals_hbm, out_hbm,
             src_r, dst_r, vals_r, rows_r, acc, semi, semg, sems):
        c_ax = lax.axis_index("c")
        s_ax = lax.axis_index("s")
        wid = c_ax * NS + s_ax
        base = wid * EPW

        def idx_load(ch, pi):
            off = base + ch * K
            pltpu.async_copy(src_hbm.at[pl.ds(off, K)], src_r.at[pi], semi.at[pi])
            pltpu.async_copy(dst_hbm.at[pl.ds(off, K)], dst_r.at[pi], semi.at[pi])
            pltpu.async_copy(vals_hbm.at[pl.ds(off, K)], vals_r.at[pi], semi.at[pi])

        def idx_wait(ch, pi):
            off = base + ch * K
            pltpu.make_async_copy(src_hbm.at[pl.ds(off, K)], src_r.at[pi], semi.at[pi]).wait()
            pltpu.make_async_copy(dst_hbm.at[pl.ds(off, K)], dst_r.at[pi], semi.at[pi]).wait()
            pltpu.make_async_copy(vals_hbm.at[pl.ds(off, K)], vals_r.at[pi], semi.at[pi]).wait()

        def gather_start(pr, pi):
            pltpu.async_copy(x_hbm.at[src_r.at[pi]], rows_r.at[pr], semg.at[pr])

        def gather_wait(pr, pi):
            pltpu.make_async_copy(x_hbm.at[src_r.at[pi]], rows_r.at[pr], semg.at[pr]).wait()

        def scat_start(pr, pi):
            pltpu.async_copy(rows_r.at[pr], acc.at[dst_r.at[pi]], sems.at[pr], add=True)

        def scat_wait(pr, pi):
            pltpu.make_async_copy(rows_r.at[pr], acc.at[dst_r.at[pi]], sems.at[pr]).wait()

        for ch in range(LA):
            idx_load(ch, ch)

        def _zb(i, carry):
            rows_r[0, i // 8, pl.ds((i % 8) * 16, 16)] = jnp.zeros((16,), jnp.float32)
            return carry
        lax.fori_loop(0, ZR * 8, _zb, 0)
        for t in range(-(-NZCH // NS)):
            q = s_ax + NS * t

            @pl.when(q < NZCH)
            def _():
                pltpu.sync_copy(rows_r.at[0], acc.at[pl.ds(q * ZR, ZR)])
        plsc.subcore_barrier()

        idx_wait(0, 0)
        gather_start(0, 0)

        def process(ch, pr, pi):
            @pl.when(ch + LA < NCHUNK)
            def _():
                idx_load(ch + LA, (pi + LA) % NI)

            @pl.when(ch + 1 < NCHUNK)
            def _():
                @pl.when(ch + 1 >= NR)
                def _():
                    scat_wait((pr + 1) % NR, (pi + 1 + NI - NR) % NI)
                idx_wait(ch + 1, (pi + 1) % NI)
                gather_start((pr + 1) % NR, (pi + 1) % NI)

            gather_wait(pr, pi)

            def sg(g, carry):
                vv = vals_r[pi, pl.ds(g * 16, 16)]
                for j in range(16):
                    val = vv[j]
                    r = g * 16 + j
                    for f in range(D // 16):
                        sl = pl.ds(f * 16, 16)
                        rows_r[pr, r, sl] = rows_r[pr, r, sl] * val
                return carry
            lax.fori_loop(0, K // 16, sg, 0)

            scat_start(pr, pi)

        nfull = (NCHUNK // STEP) * STEP

        def _main(i, carry):
            c0 = i * STEP
            for u in range(STEP):
                process(c0 + u, u % NR, u % NI)
            return carry
        lax.fori_loop(0, NCHUNK // STEP, _main, 0)
        for ch in range(nfull, NCHUNK):
            process(ch, ch % NR, ch % NI)

        for ch in range(NCHUNK - NR, NCHUNK):
            scat_wait(ch % NR, ch % NI)

        plsc.subcore_barrier()

        for t in range(-(-NZCH // NS)):
            q = s_ax + NS * t

            @pl.when(q < NZCH)
            def _():
                pltpu.sync_copy(acc.at[pl.ds(q * ZR, ZR)],
                                out_hbm.at[c_ax, pl.ds(q * ZR, ZR)])

    return spmm(x, src1d, dst1d, values)


BLK = 1000


def _tc_linear_body(acc_ref, w_ref, b_ref, o_ref):
    a = acc_ref[0] + acc_ref[1]
    o_ref[...] = lax.dot_general(
        a, w_ref[...], (((1,), (1,)), ((), ())),
        preferred_element_type=jnp.float32) + b_ref[...]


def _tc_linear(acc2, W, b2):
    return pl.pallas_call(
        _tc_linear_body,
        grid=(N // BLK,),
        in_specs=[
            pl.BlockSpec((NC, BLK, D), lambda i: (0, i, 0)),
            pl.BlockSpec((D, D), lambda i: (0, 0)),
            pl.BlockSpec((1, D), lambda i: (0, 0)),
        ],
        out_specs=pl.BlockSpec((BLK, D), lambda i: (i, 0)),
        out_shape=jax.ShapeDtypeStruct((N, D), jnp.float32),
    )(acc2, W, b2)


def kernel(x, edge_index, values, W, b):
    ei = edge_index.astype(jnp.int32)
    acc2 = _sc_spmm(x, ei[1], ei[0], values)
    return _tc_linear(acc2, W, b.reshape(1, D))
